# Initial kernel scaffold; baseline (speedup 1.0000x reference)
import functools

import jax
import jax.numpy as jnp
from jax import lax
from jax.experimental import pallas as pl
from jax.experimental.pallas import tpu as pltpu
from jax.experimental.pallas import tpu_sc as plsc

_DIM = 32
_NC = 2
_NS = 16
_NW = _NC * _NS


@functools.lru_cache(maxsize=None)
def _make_lookup(B, b_per_w, ch):
  n_ch = b_per_w // ch
  mesh = plsc.VectorSubcoreMesh(core_axis_name="c", subcore_axis_name="s")

  @functools.partial(
      pl.kernel,
      out_type=jax.ShapeDtypeStruct((B, _DIM), jnp.float32),
      mesh=mesh,
      scratch_types=[
          pltpu.VMEM((ch,), jnp.int32),
          pltpu.VMEM((ch, _DIM), jnp.float32),
          pltpu.SemaphoreType.DMA,
      ],
  )
  def lookup(idx_hbm, table_hbm, out_hbm, idx_v, rows_v, sem):
    wid = lax.axis_index("s") * _NC + lax.axis_index("c")
    base = wid * b_per_w

    def body(g, carry):
      off = base + g * ch
      pltpu.sync_copy(idx_hbm.at[pl.ds(off, ch)], idx_v)
      pltpu.async_copy(table_hbm.at[idx_v], rows_v, sem).wait()
      pltpu.sync_copy(rows_v, out_hbm.at[pl.ds(off, ch)])
      return carry

    lax.fori_loop(0, n_ch, body, 0, unroll=False)

  return lookup


def kernel(inputs, embeddings):
  shape = inputs.shape
  flat = jnp.reshape(inputs, (-1,)).astype(jnp.int32)
  b = flat.shape[0]
  b_per_w = b // _NW
  ch = 1024
  out = _make_lookup(b, b_per_w, ch)(flat, embeddings)
  return jnp.reshape(out, shape + (_DIM,))


# speedup vs baseline: 4.8052x; 4.8052x over previous
import functools

import jax
import jax.numpy as jnp
from jax import lax
from jax.experimental import pallas as pl
from jax.experimental.pallas import tpu as pltpu
from jax.experimental.pallas import tpu_sc as plsc

_DIM = 32
_NC = 2
_NS = 16
_NW = _NC * _NS


@functools.lru_cache(maxsize=None)
def _make_lookup(B, b_per_w, ch):
  n_ch = b_per_w // ch
  mesh = plsc.VectorSubcoreMesh(core_axis_name="c", subcore_axis_name="s")

  @functools.partial(
      pl.kernel,
      out_type=jax.ShapeDtypeStruct((B, _DIM), jnp.float32),
      mesh=mesh,
      scratch_types=[
          pltpu.VMEM((ch,), jnp.int32),
          pltpu.VMEM((ch, _DIM), jnp.float32),
          pltpu.SemaphoreType.DMA,
      ],
      compiler_params=pltpu.CompilerParams(use_tc_tiling_on_sc=False),
  )
  def lookup(idx_hbm, table_hbm, out_hbm, idx_v, rows_v, sem):
    wid = lax.axis_index("s") * _NC + lax.axis_index("c")
    base = wid * b_per_w

    def body(g, carry):
      off = base + g * ch
      pltpu.sync_copy(idx_hbm.at[pl.ds(off, ch)], idx_v)
      pltpu.async_copy(table_hbm.at[idx_v], rows_v, sem).wait()
      pltpu.sync_copy(rows_v, out_hbm.at[pl.ds(off, ch)])
      return carry

    lax.fori_loop(0, n_ch, body, 0, unroll=False)

  return lookup


def kernel(inputs, embeddings):
  shape = inputs.shape
  flat = jnp.reshape(inputs, (-1,)).astype(jnp.int32)
  b = flat.shape[0]
  b_per_w = b // _NW
  ch = 1024
  out = _make_lookup(b, b_per_w, ch)(flat, embeddings)
  return jnp.reshape(out, shape + (_DIM,))


# speedup vs baseline: 5.0480x; 1.0505x over previous
import functools

import jax
import jax.numpy as jnp
from jax import lax
from jax.experimental import pallas as pl
from jax.experimental.pallas import tpu as pltpu
from jax.experimental.pallas import tpu_sc as plsc

_DIM = 32
_NC = 2
_NS = 16
_NW = _NC * _NS


@functools.lru_cache(maxsize=None)
def _make_lookup(B, b_per_w, ch):
  n_ch = b_per_w // ch
  assert n_ch >= 2
  mesh = plsc.VectorSubcoreMesh(core_axis_name="c", subcore_axis_name="s")

  @functools.partial(
      pl.kernel,
      out_type=jax.ShapeDtypeStruct((B, _DIM), jnp.float32),
      mesh=mesh,
      scratch_types=[
          pltpu.VMEM((2, ch), jnp.int32),
          pltpu.VMEM((2, ch, _DIM), jnp.float32),
          pltpu.SemaphoreType.DMA((2,)),
          pltpu.SemaphoreType.DMA((2,)),
          pltpu.SemaphoreType.DMA((2,)),
      ],
      compiler_params=pltpu.CompilerParams(use_tc_tiling_on_sc=False),
  )
  def lookup(idx_hbm, table_hbm, out_hbm, idx_v, rows_v, i_sem, g_sem, o_sem):
    wid = lax.axis_index("s") * _NC + lax.axis_index("c")
    base = wid * b_per_w

    def idx_start(g, b):
      pltpu.async_copy(
          idx_hbm.at[pl.ds(base + g * ch, ch)], idx_v.at[b], i_sem.at[b])

    def idx_wait(b):
      pltpu.make_async_copy(
          idx_hbm.at[pl.ds(base, ch)], idx_v.at[b], i_sem.at[b]).wait()

    def gather_start(b):
      pltpu.async_copy(table_hbm.at[idx_v.at[b]], rows_v.at[b], g_sem.at[b])

    def gather_wait(b):
      pltpu.make_async_copy(
          table_hbm.at[idx_v.at[b]], rows_v.at[b], g_sem.at[b]).wait()

    def out_start(g, b):
      pltpu.async_copy(
          rows_v.at[b], out_hbm.at[pl.ds(base + g * ch, ch)], o_sem.at[b])

    def out_wait(b):
      pltpu.make_async_copy(
          rows_v.at[b], out_hbm.at[pl.ds(base, ch)], o_sem.at[b]).wait()

    idx_start(0, 0)
    idx_start(1, 1)
    idx_wait(0)
    gather_start(0)

    def body(g, carry):
      b = g % 2
      nb = (g + 1) % 2

      @pl.when(g + 1 < n_ch)
      def _():
        idx_wait(nb)

        @pl.when(g >= 1)
        def _():
          out_wait(nb)

        gather_start(nb)

      gather_wait(b)

      @pl.when(g + 2 < n_ch)
      def _():
        idx_start(g + 2, b)

      out_start(g, b)
      return carry

    lax.fori_loop(0, n_ch, body, 0, unroll=False)
    out_wait((n_ch - 1) % 2)

  return lookup


def kernel(inputs, embeddings):
  shape = inputs.shape
  flat = jnp.reshape(inputs, (-1,)).astype(jnp.int32)
  b = flat.shape[0]
  b_per_w = b // _NW
  ch = 1024
  out = _make_lookup(b, b_per_w, ch)(flat, embeddings)
  return jnp.reshape(out, shape + (_DIM,))
